# baseline (speedup 1.0000x reference)
import jax
import jax.numpy as jnp
from jax import lax
from jax.experimental import pallas as pl
from jax.experimental.pallas import tpu as pltpu
from jax.experimental.pallas import tpu_sc as plsc

N = 10000
E = 320000
H = 8
HID = 32
OUT = 64

NT = 16
EPT = E // NT
C = 160
NCHUNK = EPT // C
ZB = 624
GW = 2 * HID

NB = 1000


def _head_onehot(dtype=jnp.float32):
    t = lax.broadcasted_iota(jnp.int32, (H * HID, H), 0)
    h = lax.broadcasted_iota(jnp.int32, (H * HID, H), 1)
    return (t // HID == h).astype(dtype)


def _col_expand():
    h = lax.broadcasted_iota(jnp.int32, (2, GW), 0)
    t = lax.broadcasted_iota(jnp.int32, (2, GW), 1)
    return (t // HID == h).astype(jnp.float32)


def _head_sum():
    t = lax.broadcasted_iota(jnp.int32, (GW, HID), 0)
    d = lax.broadcasted_iota(jnp.int32, (GW, HID), 1)
    return (t % HID == d).astype(jnp.float32)


def _emit_feat_el_er(feat, al_ref, ar_ref, feat_ref, el_ref, er_ref):
    feat_ref[...] = feat
    G = _head_onehot()
    el8 = jnp.dot(feat * al_ref[...], G, preferred_element_type=jnp.float32)
    er8 = jnp.dot(feat * ar_ref[...], G, preferred_element_type=jnp.float32)
    for g in range(4):
        el_ref[g] = el8[:, 2 * g:2 * g + 2]
        er_ref[g] = er8[:, 2 * g:2 * g + 2]


def _pre_body(h_ref, w_ref, al_ref, ar_ref, feat_ref, el_ref, er_ref):
    feat = jnp.dot(h_ref[...], w_ref[...], preferred_element_type=jnp.float32)
    _emit_feat_el_er(feat, al_ref, ar_ref, feat_ref, el_ref, er_ref)


def _dense_out_common():
    return (
        [
            pl.BlockSpec((NB, H * HID), lambda i: (i, 0)),
            pl.BlockSpec((4, NB, 2), lambda i: (0, i, 0)),
            pl.BlockSpec((4, NB, 2), lambda i: (0, i, 0)),
        ],
        [
            jax.ShapeDtypeStruct((N, H * HID), jnp.float32),
            jax.ShapeDtypeStruct((4, N, 2), jnp.float32),
            jax.ShapeDtypeStruct((4, N, 2), jnp.float32),
        ],
    )


def _dense_pre(h, W, alf, arf):
    F = h.shape[1]
    out_specs, out_shape = _dense_out_common()
    return pl.pallas_call(
        _pre_body,
        grid=(N // NB,),
        in_specs=[
            pl.BlockSpec((NB, F), lambda i: (i, 0)),
            pl.BlockSpec((F, H * HID), lambda i: (0, 0)),
            pl.BlockSpec((1, H * HID), lambda i: (0, 0)),
            pl.BlockSpec((1, H * HID), lambda i: (0, 0)),
        ],
        out_specs=out_specs,
        out_shape=out_shape,
    )(h, W, alf, arf)


def _normalize_hsum(acc_ref, s_ref, b_ref):
    E2 = _col_expand()
    K = _head_sum()
    hs = jnp.zeros((acc_ref.shape[2], HID), jnp.float32)
    for c in range(2):
        for p in range(2):
            g = 2 * c + p
            s2 = s_ref[c, p][:, 0:2]
            rec = 1.0 / jnp.maximum(s2, 1e-9)
            recx = jnp.dot(rec, E2, preferred_element_type=jnp.float32)
            scaled = acc_ref[c, p] * recx + b_ref[0, g * GW:(g + 1) * GW]
            hs = hs + jnp.dot(scaled, K, preferred_element_type=jnp.float32)
    return jnp.maximum(hs, 0.01 * hs)


def _mid_body(acc_ref, s_ref, b_ref, w_ref, al_ref, ar_ref,
              feat_ref, el_ref, er_ref):
    hb = _normalize_hsum(acc_ref, s_ref, b_ref)
    feat = jnp.dot(hb, w_ref[...], preferred_element_type=jnp.float32)
    _emit_feat_el_er(feat, al_ref, ar_ref, feat_ref, el_ref, er_ref)


def _acc_in_specs():
    return [
        pl.BlockSpec((2, 2, NB, GW), lambda i: (0, 0, i, 0)),
        pl.BlockSpec((2, 2, NB, 16), lambda i: (0, 0, i, 0)),
        pl.BlockSpec((1, H * HID), lambda i: (0, 0)),
    ]


def _dense_mid(acc, s, bprev, W, alf, arf):
    out_specs, out_shape = _dense_out_common()
    return pl.pallas_call(
        _mid_body,
        grid=(N // NB,),
        in_specs=_acc_in_specs() + [
            pl.BlockSpec((HID, H * HID), lambda i: (0, 0)),
            pl.BlockSpec((1, H * HID), lambda i: (0, 0)),
            pl.BlockSpec((1, H * HID), lambda i: (0, 0)),
        ],
        out_specs=out_specs,
        out_shape=out_shape,
    )(acc, s, bprev, W, alf, arf)


def _fin_body(acc_ref, s_ref, b_ref, wm_ref, bm_ref, out_ref):
    hb = _normalize_hsum(acc_ref, s_ref, b_ref)
    out_ref[...] = (jnp.dot(hb, wm_ref[...], preferred_element_type=jnp.float32)
                    + bm_ref[...])


def _dense_fin(acc, s, bprev, Wm, bmf):
    return pl.pallas_call(
        _fin_body,
        grid=(N // NB,),
        in_specs=_acc_in_specs() + [
            pl.BlockSpec((HID, OUT), lambda i: (0, 0)),
            pl.BlockSpec((1, OUT), lambda i: (0, 0)),
        ],
        out_specs=pl.BlockSpec((NB, OUT), lambda i: (i, 0)),
        out_shape=jax.ShapeDtypeStruct((N, OUT), jnp.float32),
    )(acc, s, bprev, Wm, bmf)


def _edge_body(featc, elf, erf, edges, acc_out, s_out,
               el_st, er_st, src_b, dst_b, idx_b, wf_b, rows_b,
               acc_sh, s_sh, sems, ssems):
    c = lax.axis_index("c")
    wid = lax.axis_index("s")
    zv = jnp.zeros((16,), jnp.float32)

    def _zero_wf(r, _):
        for q in range(3):
            wf_b[q][r, :] = zv
        return 0

    lax.fori_loop(0, C, _zero_wf, 0)

    def _dma_idx(i, bb):
        base = wid * EPT + i * C
        pltpu.sync_copy(edges.at[pl.ds(base, C)], src_b[bb])
        pltpu.sync_copy(edges.at[pl.ds(E + base, C)], dst_b[bb])

    def _wcomp(bb, g):
        sb, db, ib, wf = src_b[bb], dst_b[bb], idx_b[bb], wf_b[bb]

        def _wgrp(j):
            sv = sb[pl.ds(16 * j, 16)]
            dv = db[pl.ds(16 * j, 16)]
            ib[pl.ds(16 * j, 16)] = sv * 4 + g
            rowv = lax.broadcasted_iota(jnp.int32, (16,), 0) + 16 * j
            sv2 = sv * 2
            dv2 = dv * 2
            for hl in range(2):
                ev = plsc.load_gather(el_st, [sv2 + hl])
                rv = plsc.load_gather(er_st, [dv2 + hl])
                x = ev + rv
                x = jnp.maximum(x, 0.2 * x)
                hlv = jnp.full((16,), hl, jnp.int32)
                plsc.store_scatter(wf, [rowv, hlv], jnp.exp(x))

        plsc.parallel_loop(0, C // 16, unroll=2)(_wgrp)

    def _scale(bb):
        wf, rb = wf_b[bb], rows_b[bb]

        def _srow(r):
            wv = wf[r, :]
            for hl in range(2):
                ws = wv[hl]
                for k in range(2):
                    col = hl * HID + 16 * k
                    rb[r, pl.ds(col, 16)] = rb[r, pl.ds(col, 16)] * ws

        plsc.parallel_loop(0, C, unroll=8)(_srow)

    def _scatter(bb):
        pltpu.async_copy(rows_b[bb], acc_sh.at[dst_b[bb]], ssems[bb], add=True)
        pltpu.async_copy(wf_b[bb], s_sh.at[dst_b[bb]], ssems[bb], add=True)

    def _drain(bb):
        pltpu.make_async_copy(rows_b[bb], acc_sh.at[dst_b[bb]],
                              ssems[bb]).wait()
        pltpu.make_async_copy(wf_b[bb], s_sh.at[dst_b[bb]], ssems[bb]).wait()

    def _prefetch(i, bb, g):
        _dma_idx(i, bb)
        _wcomp(bb, g)
        pltpu.async_copy(featc.at[idx_b[bb]], rows_b[bb], sems[bb])

    def _wait_gather(bb):
        pltpu.make_async_copy(featc.at[idx_b[bb]], rows_b[bb],
                              sems[bb]).wait()

    for p in range(2):
        g = c * 2 + p

        pltpu.sync_copy(elf.at[pl.ds(g * 2 * N, 2 * N)], el_st)
        pltpu.sync_copy(erf.at[pl.ds(g * 2 * N, 2 * N)], er_st)

        def _zero_rows(r, _):
            for k in range(GW // 16):
                rows_b[0][r, pl.ds(16 * k, 16)] = zv
            wf_b[0][r, :] = zv
            return 0

        lax.fori_loop(0, C, _zero_rows, 0)
        for off, sz in ((0, 160), (160, 160), (320, 160), (480, 144)):
            pltpu.sync_copy(rows_b[0].at[pl.ds(0, sz)],
                            acc_sh.at[pl.ds(wid * ZB + off, sz)])
            pltpu.sync_copy(wf_b[0].at[pl.ds(0, sz)],
                            s_sh.at[pl.ds(wid * ZB + off, sz)])

        @pl.when(wid < 2)
        def _():
            pltpu.sync_copy(rows_b[0].at[pl.ds(0, 8)],
                            acc_sh.at[pl.ds(NT * ZB + wid * 8, 8)])
            pltpu.sync_copy(wf_b[0].at[pl.ds(0, 8)],
                            s_sh.at[pl.ds(NT * ZB + wid * 8, 8)])

        plsc.subcore_barrier()

        _prefetch(0, 0, g)

        def _pos(i, b, drain):
            if drain:
                _drain((b + 1) % 3)
            _prefetch(i + 1, (b + 1) % 3, g)
            _wait_gather(b)
            _scale(b)
            _scatter(b)

        _pos(0, 0, False)
        _pos(1, 1, False)

        def _triple(t, _):
            i = 3 * t + 2
            _pos(i, 2, True)
            _pos(i + 1, 0, True)
            _pos(i + 2, 1, True)
            return 0

        lax.fori_loop(0, (NCHUNK - 5) // 3, _triple, 0)
        _pos(NCHUNK - 3, (NCHUNK - 3) % 3, True)
        _pos(NCHUNK - 2, (NCHUNK - 2) % 3, True)
        lb = (NCHUNK - 1) % 3
        _drain((lb + 1) % 3)
        _wait_gather(lb)
        _scale(lb)
        _scatter(lb)
        _drain((lb + 2) % 3)
        _drain(lb)
        plsc.subcore_barrier()

        r0 = wid * ZB
        pltpu.sync_copy(acc_sh.at[pl.ds(r0, ZB)],
                        acc_out.at[c, p, pl.ds(r0, ZB)])
        pltpu.sync_copy(s_sh.at[pl.ds(r0, ZB)],
                        s_out.at[c, p, pl.ds(r0, ZB)])

        @pl.when(wid < 2)
        def _():
            r1 = NT * ZB + wid * 8
            pltpu.sync_copy(acc_sh.at[pl.ds(r1, 8)],
                            acc_out.at[c, p, pl.ds(r1, 8)])
            pltpu.sync_copy(s_sh.at[pl.ds(r1, 8)],
                            s_out.at[c, p, pl.ds(r1, 8)])

        plsc.subcore_barrier()


def _edge_call(featc, elf, erf, edges):
    mesh = plsc.VectorSubcoreMesh(core_axis_name="c", subcore_axis_name="s")
    fn = pl.kernel(
        _edge_body,
        out_type=[
            jax.ShapeDtypeStruct((2, 2, N, GW), jnp.float32),
            jax.ShapeDtypeStruct((2, 2, N, 16), jnp.float32),
        ],
        mesh=mesh,
        compiler_params=pltpu.CompilerParams(needs_layout_passes=False, use_tc_tiling_on_sc=False),
        scratch_types=[
            pltpu.VMEM((2 * N,), jnp.float32),
            pltpu.VMEM((2 * N,), jnp.float32),
            [pltpu.VMEM((C,), jnp.int32)] * 3,
            [pltpu.VMEM((C,), jnp.int32)] * 3,
            [pltpu.VMEM((C,), jnp.int32)] * 3,
            [pltpu.VMEM((C, 16), jnp.float32)] * 3,
            [pltpu.VMEM((C, GW), jnp.float32)] * 3,
            pltpu.VMEM_SHARED((N, GW), jnp.float32),
            pltpu.VMEM_SHARED((N, 16), jnp.float32),
            [pltpu.SemaphoreType.DMA] * 3,
            [pltpu.SemaphoreType.DMA] * 3,
        ],
    )
    return fn(featc, elf, erf, edges)


def kernel(inputs, edge_index, W1, al1, ar1, b1, W2, al2, ar2, b2,
           W3, al3, ar3, b3, Wm, bm):
    edges = edge_index.reshape(2 * E)

    def edge(feat, el2, er2):
        return _edge_call(feat.reshape(4 * N, GW), el2.reshape(8 * N),
                          er2.reshape(8 * N), edges)

    feat, el2, er2 = _dense_pre(inputs, W1, al1.reshape(1, H * HID),
                                ar1.reshape(1, H * HID))
    acc, s = edge(feat, el2, er2)

    feat, el2, er2 = _dense_mid(acc, s, b1.reshape(1, H * HID), W2,
                                al2.reshape(1, H * HID), ar2.reshape(1, H * HID))
    acc, s = edge(feat, el2, er2)

    feat, el2, er2 = _dense_mid(acc, s, b2.reshape(1, H * HID), W3,
                                al3.reshape(1, H * HID), ar3.reshape(1, H * HID))
    acc, s = edge(feat, el2, er2)

    return _dense_fin(acc, s, b3.reshape(1, H * HID), Wm, bm.reshape(1, OUT))


# speedup vs baseline: 1.0708x; 1.0708x over previous
import jax
import jax.numpy as jnp
from jax import lax
from jax.experimental import pallas as pl
from jax.experimental.pallas import tpu as pltpu
from jax.experimental.pallas import tpu_sc as plsc

N = 10000
E = 320000
H = 8
HID = 32
OUT = 64

NT = 16
EPT = E // NT
C = 160
NCHUNK = EPT // C
ZB = 624
GW = 2 * HID

NB = 1000


def _head_onehot(dtype=jnp.float32):
    t = lax.broadcasted_iota(jnp.int32, (H * HID, H), 0)
    h = lax.broadcasted_iota(jnp.int32, (H * HID, H), 1)
    return (t // HID == h).astype(dtype)


def _col_expand():
    h = lax.broadcasted_iota(jnp.int32, (2, GW), 0)
    t = lax.broadcasted_iota(jnp.int32, (2, GW), 1)
    return (t // HID == h).astype(jnp.float32)


def _head_sum():
    t = lax.broadcasted_iota(jnp.int32, (GW, HID), 0)
    d = lax.broadcasted_iota(jnp.int32, (GW, HID), 1)
    return (t % HID == d).astype(jnp.float32)


def _emit_feat_el_er(feat, al_ref, ar_ref, feat_ref, el_ref, er_ref):
    feat_ref[...] = feat
    G = _head_onehot()
    el8 = jnp.dot(feat * al_ref[...], G, preferred_element_type=jnp.float32)
    er8 = jnp.dot(feat * ar_ref[...], G, preferred_element_type=jnp.float32)
    for g in range(4):
        el_ref[g] = el8[:, 2 * g:2 * g + 2]
        er_ref[g] = er8[:, 2 * g:2 * g + 2]


def _pre_body(h_ref, w_ref, al_ref, ar_ref, feat_ref, el_ref, er_ref):
    feat = jnp.dot(h_ref[...], w_ref[...], preferred_element_type=jnp.float32)
    _emit_feat_el_er(feat, al_ref, ar_ref, feat_ref, el_ref, er_ref)


def _dense_out_common():
    return (
        [
            pl.BlockSpec((NB, H * HID), lambda i: (i, 0)),
            pl.BlockSpec((4, NB, 2), lambda i: (0, i, 0)),
            pl.BlockSpec((4, NB, 2), lambda i: (0, i, 0)),
        ],
        [
            jax.ShapeDtypeStruct((N, H * HID), jnp.float32),
            jax.ShapeDtypeStruct((4, N, 2), jnp.float32),
            jax.ShapeDtypeStruct((4, N, 2), jnp.float32),
        ],
    )


def _dense_pre(h, W, alf, arf):
    F = h.shape[1]
    out_specs, out_shape = _dense_out_common()
    return pl.pallas_call(
        _pre_body,
        grid=(N // NB,),
        in_specs=[
            pl.BlockSpec((NB, F), lambda i: (i, 0)),
            pl.BlockSpec((F, H * HID), lambda i: (0, 0)),
            pl.BlockSpec((1, H * HID), lambda i: (0, 0)),
            pl.BlockSpec((1, H * HID), lambda i: (0, 0)),
        ],
        out_specs=out_specs,
        out_shape=out_shape,
    )(h, W, alf, arf)


def _normalize_hsum(acc_ref, s_ref, b_ref):
    E2 = _col_expand()
    K = _head_sum()
    hs = jnp.zeros((acc_ref.shape[2], HID), jnp.float32)
    for c in range(2):
        for p in range(2):
            g = 2 * c + p
            s2 = s_ref[c, p][:, 0:2]
            rec = 1.0 / jnp.maximum(s2, 1e-9)
            recx = jnp.dot(rec, E2, preferred_element_type=jnp.float32)
            scaled = acc_ref[c, p] * recx + b_ref[0, g * GW:(g + 1) * GW]
            hs = hs + jnp.dot(scaled, K, preferred_element_type=jnp.float32)
    return jnp.maximum(hs, 0.01 * hs)


def _mid_body(acc_ref, s_ref, b_ref, w_ref, al_ref, ar_ref,
              feat_ref, el_ref, er_ref):
    hb = _normalize_hsum(acc_ref, s_ref, b_ref)
    feat = jnp.dot(hb, w_ref[...], preferred_element_type=jnp.float32)
    _emit_feat_el_er(feat, al_ref, ar_ref, feat_ref, el_ref, er_ref)


def _acc_in_specs():
    return [
        pl.BlockSpec((2, 2, NB, GW), lambda i: (0, 0, i, 0)),
        pl.BlockSpec((2, 2, NB, 16), lambda i: (0, 0, i, 0)),
        pl.BlockSpec((1, H * HID), lambda i: (0, 0)),
    ]


def _dense_mid(acc, s, bprev, W, alf, arf):
    out_specs, out_shape = _dense_out_common()
    return pl.pallas_call(
        _mid_body,
        grid=(N // NB,),
        in_specs=_acc_in_specs() + [
            pl.BlockSpec((HID, H * HID), lambda i: (0, 0)),
            pl.BlockSpec((1, H * HID), lambda i: (0, 0)),
            pl.BlockSpec((1, H * HID), lambda i: (0, 0)),
        ],
        out_specs=out_specs,
        out_shape=out_shape,
    )(acc, s, bprev, W, alf, arf)


def _fin_body(acc_ref, s_ref, b_ref, wm_ref, bm_ref, out_ref):
    hb = _normalize_hsum(acc_ref, s_ref, b_ref)
    out_ref[...] = (jnp.dot(hb, wm_ref[...], preferred_element_type=jnp.float32)
                    + bm_ref[...])


def _dense_fin(acc, s, bprev, Wm, bmf):
    return pl.pallas_call(
        _fin_body,
        grid=(N // NB,),
        in_specs=_acc_in_specs() + [
            pl.BlockSpec((HID, OUT), lambda i: (0, 0)),
            pl.BlockSpec((1, OUT), lambda i: (0, 0)),
        ],
        out_specs=pl.BlockSpec((NB, OUT), lambda i: (i, 0)),
        out_shape=jax.ShapeDtypeStruct((N, OUT), jnp.float32),
    )(acc, s, bprev, Wm, bmf)


def _edge_body(featc, elf, erf, edges, acc_out, s_out,
               el_st, er_st, ed_b, dst_b, idx_b, wf_b, rows_b,
               acc_sh, s_sh, sems, ssems):
    c = lax.axis_index("c")
    wid = lax.axis_index("s")
    zv = jnp.zeros((16,), jnp.float32)

    def _zero_wf(r, _):
        for q in range(3):
            wf_b[q][r, :] = zv
        return 0

    lax.fori_loop(0, C, _zero_wf, 0)

    def _dma_idx(i, bb):
        base = wid * EPT + i * C
        pltpu.sync_copy(edges.at[pl.ds(2 * base, 2 * C)], ed_b[bb])

    def _wcomp(bb, g):
        eb, db, ib, wf = ed_b[bb], dst_b[bb], idx_b[bb], wf_b[bb]

        def _wgrp(j):
            rv2 = (lax.broadcasted_iota(jnp.int32, (16,), 0) + 16 * j) * 2
            sv = plsc.load_gather(eb, [rv2])
            dv = plsc.load_gather(eb, [rv2 + 1])
            db[pl.ds(16 * j, 16)] = dv
            ib[pl.ds(16 * j, 16)] = sv * 4 + g
            rowv = lax.broadcasted_iota(jnp.int32, (16,), 0) + 16 * j
            sv2 = sv * 2
            dv2 = dv * 2
            for hl in range(2):
                ev = plsc.load_gather(el_st, [sv2 + hl])
                rv = plsc.load_gather(er_st, [dv2 + hl])
                x = ev + rv
                x = jnp.maximum(x, 0.2 * x)
                hlv = jnp.full((16,), hl, jnp.int32)
                plsc.store_scatter(wf, [rowv, hlv], jnp.exp(x))

        plsc.parallel_loop(0, C // 16, unroll=2)(_wgrp)

    def _scale(bb):
        wf, rb = wf_b[bb], rows_b[bb]

        def _srow(r):
            wv = wf[r, :]
            for hl in range(2):
                ws = wv[hl]
                for k in range(2):
                    col = hl * HID + 16 * k
                    rb[r, pl.ds(col, 16)] = rb[r, pl.ds(col, 16)] * ws

        plsc.parallel_loop(0, C, unroll=8)(_srow)

    def _scatter(bb):
        pltpu.async_copy(rows_b[bb], acc_sh.at[dst_b[bb]], ssems[bb], add=True)
        pltpu.async_copy(wf_b[bb], s_sh.at[dst_b[bb]], ssems[bb], add=True)

    def _drain(bb):
        pltpu.make_async_copy(rows_b[bb], acc_sh.at[dst_b[bb]],
                              ssems[bb]).wait()
        pltpu.make_async_copy(wf_b[bb], s_sh.at[dst_b[bb]], ssems[bb]).wait()

    def _prefetch(i, bb, g):
        _dma_idx(i, bb)
        _wcomp(bb, g)
        pltpu.async_copy(featc.at[idx_b[bb]], rows_b[bb], sems[bb])

    def _wait_gather(bb):
        pltpu.make_async_copy(featc.at[idx_b[bb]], rows_b[bb],
                              sems[bb]).wait()

    for p in range(2):
        g = c * 2 + p

        pltpu.sync_copy(elf.at[pl.ds(g * 2 * N, 2 * N)], el_st)
        pltpu.sync_copy(erf.at[pl.ds(g * 2 * N, 2 * N)], er_st)

        def _zero_rows(r, _):
            for k in range(GW // 16):
                rows_b[0][r, pl.ds(16 * k, 16)] = zv
            wf_b[0][r, :] = zv
            return 0

        lax.fori_loop(0, C, _zero_rows, 0)
        for off, sz in ((0, 160), (160, 160), (320, 160), (480, 144)):
            pltpu.sync_copy(rows_b[0].at[pl.ds(0, sz)],
                            acc_sh.at[pl.ds(wid * ZB + off, sz)])
            pltpu.sync_copy(wf_b[0].at[pl.ds(0, sz)],
                            s_sh.at[pl.ds(wid * ZB + off, sz)])

        @pl.when(wid < 2)
        def _():
            pltpu.sync_copy(rows_b[0].at[pl.ds(0, 8)],
                            acc_sh.at[pl.ds(NT * ZB + wid * 8, 8)])
            pltpu.sync_copy(wf_b[0].at[pl.ds(0, 8)],
                            s_sh.at[pl.ds(NT * ZB + wid * 8, 8)])

        plsc.subcore_barrier()

        _prefetch(0, 0, g)

        def _pos(i, b, drain):
            if drain:
                _drain((b + 1) % 3)
            _prefetch(i + 1, (b + 1) % 3, g)
            _wait_gather(b)
            _scale(b)
            _scatter(b)

        _pos(0, 0, False)
        _pos(1, 1, False)

        def _triple(t, _):
            i = 3 * t + 2
            _pos(i, 2, True)
            _pos(i + 1, 0, True)
            _pos(i + 2, 1, True)
            return 0

        lax.fori_loop(0, (NCHUNK - 5) // 3, _triple, 0)
        _pos(NCHUNK - 3, (NCHUNK - 3) % 3, True)
        _pos(NCHUNK - 2, (NCHUNK - 2) % 3, True)
        lb = (NCHUNK - 1) % 3
        _drain((lb + 1) % 3)
        _wait_gather(lb)
        _scale(lb)
        _scatter(lb)
        _drain((lb + 2) % 3)
        _drain(lb)
        plsc.subcore_barrier()

        r0 = wid * ZB
        pltpu.sync_copy(acc_sh.at[pl.ds(r0, ZB)],
                        acc_out.at[c, p, pl.ds(r0, ZB)])
        pltpu.sync_copy(s_sh.at[pl.ds(r0, ZB)],
                        s_out.at[c, p, pl.ds(r0, ZB)])

        @pl.when(wid < 2)
        def _():
            r1 = NT * ZB + wid * 8
            pltpu.sync_copy(acc_sh.at[pl.ds(r1, 8)],
                            acc_out.at[c, p, pl.ds(r1, 8)])
            pltpu.sync_copy(s_sh.at[pl.ds(r1, 8)],
                            s_out.at[c, p, pl.ds(r1, 8)])

        plsc.subcore_barrier()


def _edge_call(featc, elf, erf, edges):
    mesh = plsc.VectorSubcoreMesh(core_axis_name="c", subcore_axis_name="s")
    fn = pl.kernel(
        _edge_body,
        out_type=[
            jax.ShapeDtypeStruct((2, 2, N, GW), jnp.float32),
            jax.ShapeDtypeStruct((2, 2, N, 16), jnp.float32),
        ],
        mesh=mesh,
        compiler_params=pltpu.CompilerParams(needs_layout_passes=False, use_tc_tiling_on_sc=False),
        scratch_types=[
            pltpu.VMEM((2 * N,), jnp.float32),
            pltpu.VMEM((2 * N,), jnp.float32),
            [pltpu.VMEM((2 * C,), jnp.int32)] * 3,
            [pltpu.VMEM((C,), jnp.int32)] * 3,
            [pltpu.VMEM((C,), jnp.int32)] * 3,
            [pltpu.VMEM((C, 16), jnp.float32)] * 3,
            [pltpu.VMEM((C, GW), jnp.float32)] * 3,
            pltpu.VMEM_SHARED((N, GW), jnp.float32),
            pltpu.VMEM_SHARED((N, 16), jnp.float32),
            [pltpu.SemaphoreType.DMA] * 3,
            [pltpu.SemaphoreType.DMA] * 3,
        ],
    )
    return fn(featc, elf, erf, edges)


def kernel(inputs, edge_index, W1, al1, ar1, b1, W2, al2, ar2, b2,
           W3, al3, ar3, b3, Wm, bm):
    edges = edge_index.T.reshape(2 * E)

    def edge(feat, el2, er2):
        return _edge_call(feat.reshape(4 * N, GW), el2.reshape(8 * N),
                          er2.reshape(8 * N), edges)

    feat, el2, er2 = _dense_pre(inputs, W1, al1.reshape(1, H * HID),
                                ar1.reshape(1, H * HID))
    acc, s = edge(feat, el2, er2)

    feat, el2, er2 = _dense_mid(acc, s, b1.reshape(1, H * HID), W2,
                                al2.reshape(1, H * HID), ar2.reshape(1, H * HID))
    acc, s = edge(feat, el2, er2)

    feat, el2, er2 = _dense_mid(acc, s, b2.reshape(1, H * HID), W3,
                                al3.reshape(1, H * HID), ar3.reshape(1, H * HID))
    acc, s = edge(feat, el2, er2)

    return _dense_fin(acc, s, b3.reshape(1, H * HID), Wm, bm.reshape(1, OUT))


# speedup vs baseline: 1.0800x; 1.0086x over previous
import jax
import jax.numpy as jnp
from jax import lax
from jax.experimental import pallas as pl
from jax.experimental.pallas import tpu as pltpu
from jax.experimental.pallas import tpu_sc as plsc

N = 10000
E = 320000
H = 8
HID = 32
OUT = 64

NT = 16
EPT = E // NT
C = 160
NCHUNK = EPT // C
ZB = 624
GW = 2 * HID

NB = 2000


def _head_onehot(dtype=jnp.float32):
    t = lax.broadcasted_iota(jnp.int32, (H * HID, H), 0)
    h = lax.broadcasted_iota(jnp.int32, (H * HID, H), 1)
    return (t // HID == h).astype(dtype)


def _col_expand():
    h = lax.broadcasted_iota(jnp.int32, (2, GW), 0)
    t = lax.broadcasted_iota(jnp.int32, (2, GW), 1)
    return (t // HID == h).astype(jnp.float32)


def _head_sum():
    t = lax.broadcasted_iota(jnp.int32, (GW, HID), 0)
    d = lax.broadcasted_iota(jnp.int32, (GW, HID), 1)
    return (t % HID == d).astype(jnp.float32)


def _emit_feat_el_er(feat, al_ref, ar_ref, feat_ref, el_ref, er_ref):
    feat_ref[...] = feat
    G = _head_onehot()
    el8 = jnp.dot(feat * al_ref[...], G, preferred_element_type=jnp.float32)
    er8 = jnp.dot(feat * ar_ref[...], G, preferred_element_type=jnp.float32)
    for g in range(4):
        el_ref[g] = el8[:, 2 * g:2 * g + 2]
        er_ref[g] = er8[:, 2 * g:2 * g + 2]


def _pre_body(h_ref, w_ref, al_ref, ar_ref, feat_ref, el_ref, er_ref):
    feat = jnp.dot(h_ref[...], w_ref[...], preferred_element_type=jnp.float32)
    _emit_feat_el_er(feat, al_ref, ar_ref, feat_ref, el_ref, er_ref)


def _dense_out_common():
    return (
        [
            pl.BlockSpec((NB, H * HID), lambda i: (i, 0)),
            pl.BlockSpec((4, NB, 2), lambda i: (0, i, 0)),
            pl.BlockSpec((4, NB, 2), lambda i: (0, i, 0)),
        ],
        [
            jax.ShapeDtypeStruct((N, H * HID), jnp.float32),
            jax.ShapeDtypeStruct((4, N, 2), jnp.float32),
            jax.ShapeDtypeStruct((4, N, 2), jnp.float32),
        ],
    )


def _dense_pre(h, W, alf, arf):
    F = h.shape[1]
    out_specs, out_shape = _dense_out_common()
    return pl.pallas_call(
        _pre_body,
        grid=(N // NB,),
        in_specs=[
            pl.BlockSpec((NB, F), lambda i: (i, 0)),
            pl.BlockSpec((F, H * HID), lambda i: (0, 0)),
            pl.BlockSpec((1, H * HID), lambda i: (0, 0)),
            pl.BlockSpec((1, H * HID), lambda i: (0, 0)),
        ],
        out_specs=out_specs,
        out_shape=out_shape,
    )(h, W, alf, arf)


def _normalize_hsum(acc_ref, s_ref, b_ref):
    E2 = _col_expand()
    K = _head_sum()
    hs = jnp.zeros((acc_ref.shape[2], HID), jnp.float32)
    for c in range(2):
        for p in range(2):
            g = 2 * c + p
            s2 = s_ref[c, p][:, 0:2]
            rec = 1.0 / jnp.maximum(s2, 1e-9)
            recx = jnp.dot(rec, E2, preferred_element_type=jnp.float32)
            scaled = acc_ref[c, p] * recx + b_ref[0, g * GW:(g + 1) * GW]
            hs = hs + jnp.dot(scaled, K, preferred_element_type=jnp.float32)
    return jnp.maximum(hs, 0.01 * hs)


def _mid_body(acc_ref, s_ref, b_ref, w_ref, al_ref, ar_ref,
              feat_ref, el_ref, er_ref):
    hb = _normalize_hsum(acc_ref, s_ref, b_ref)
    feat = jnp.dot(hb, w_ref[...], preferred_element_type=jnp.float32)
    _emit_feat_el_er(feat, al_ref, ar_ref, feat_ref, el_ref, er_ref)


def _acc_in_specs():
    return [
        pl.BlockSpec((2, 2, NB, GW), lambda i: (0, 0, i, 0)),
        pl.BlockSpec((2, 2, NB, 16), lambda i: (0, 0, i, 0)),
        pl.BlockSpec((1, H * HID), lambda i: (0, 0)),
    ]


def _dense_mid(acc, s, bprev, W, alf, arf):
    out_specs, out_shape = _dense_out_common()
    return pl.pallas_call(
        _mid_body,
        grid=(N // NB,),
        in_specs=_acc_in_specs() + [
            pl.BlockSpec((HID, H * HID), lambda i: (0, 0)),
            pl.BlockSpec((1, H * HID), lambda i: (0, 0)),
            pl.BlockSpec((1, H * HID), lambda i: (0, 0)),
        ],
        out_specs=out_specs,
        out_shape=out_shape,
    )(acc, s, bprev, W, alf, arf)


def _fin_body(acc_ref, s_ref, b_ref, wm_ref, bm_ref, out_ref):
    hb = _normalize_hsum(acc_ref, s_ref, b_ref)
    out_ref[...] = (jnp.dot(hb, wm_ref[...], preferred_element_type=jnp.float32)
                    + bm_ref[...])


def _dense_fin(acc, s, bprev, Wm, bmf):
    return pl.pallas_call(
        _fin_body,
        grid=(N // NB,),
        in_specs=_acc_in_specs() + [
            pl.BlockSpec((HID, OUT), lambda i: (0, 0)),
            pl.BlockSpec((1, OUT), lambda i: (0, 0)),
        ],
        out_specs=pl.BlockSpec((NB, OUT), lambda i: (i, 0)),
        out_shape=jax.ShapeDtypeStruct((N, OUT), jnp.float32),
    )(acc, s, bprev, Wm, bmf)


def _edge_body(featc, elf, erf, edges, acc_out, s_out,
               el_st, er_st, ed_b, dst_b, idx_b, wf_b, rows_b,
               acc_sh, s_sh, sems, ssems):
    c = lax.axis_index("c")
    wid = lax.axis_index("s")
    zv = jnp.zeros((16,), jnp.float32)

    def _zero_wf(r, _):
        for q in range(3):
            wf_b[q][r, :] = zv
        return 0

    lax.fori_loop(0, C, _zero_wf, 0)

    def _dma_idx(i, bb):
        base = wid * EPT + i * C
        pltpu.sync_copy(edges.at[pl.ds(2 * base, 2 * C)], ed_b[bb])

    def _wcomp(bb, g):
        eb, db, ib, wf = ed_b[bb], dst_b[bb], idx_b[bb], wf_b[bb]

        def _wgrp(j):
            rv2 = (lax.broadcasted_iota(jnp.int32, (16,), 0) + 16 * j) * 2
            sv = plsc.load_gather(eb, [rv2])
            dv = plsc.load_gather(eb, [rv2 + 1])
            db[pl.ds(16 * j, 16)] = dv
            ib[pl.ds(16 * j, 16)] = sv * 4 + g
            rowv = lax.broadcasted_iota(jnp.int32, (16,), 0) + 16 * j
            sv2 = sv * 2
            dv2 = dv * 2
            for hl in range(2):
                ev = plsc.load_gather(el_st, [sv2 + hl])
                rv = plsc.load_gather(er_st, [dv2 + hl])
                x = ev + rv
                x = jnp.maximum(x, 0.2 * x)
                hlv = jnp.full((16,), hl, jnp.int32)
                plsc.store_scatter(wf, [rowv, hlv], jnp.exp(x))

        plsc.parallel_loop(0, C // 16, unroll=2)(_wgrp)

    def _scale(bb):
        wf, rb = wf_b[bb], rows_b[bb]

        def _srow(r):
            wv = wf[r, :]
            for hl in range(2):
                ws = wv[hl]
                for k in range(2):
                    col = hl * HID + 16 * k
                    rb[r, pl.ds(col, 16)] = rb[r, pl.ds(col, 16)] * ws

        plsc.parallel_loop(0, C, unroll=8)(_srow)

    def _scatter(bb):
        pltpu.async_copy(rows_b[bb], acc_sh.at[dst_b[bb]], ssems[bb], add=True)
        pltpu.async_copy(wf_b[bb], s_sh.at[dst_b[bb]], ssems[bb], add=True)

    def _drain(bb):
        pltpu.make_async_copy(rows_b[bb], acc_sh.at[dst_b[bb]],
                              ssems[bb]).wait()
        pltpu.make_async_copy(wf_b[bb], s_sh.at[dst_b[bb]], ssems[bb]).wait()

    def _prefetch(i, bb, g):
        _dma_idx(i, bb)
        _wcomp(bb, g)
        pltpu.async_copy(featc.at[idx_b[bb]], rows_b[bb], sems[bb])

    def _wait_gather(bb):
        pltpu.make_async_copy(featc.at[idx_b[bb]], rows_b[bb],
                              sems[bb]).wait()

    for p in range(2):
        g = c * 2 + p

        pltpu.sync_copy(elf.at[pl.ds(g * 2 * N, 2 * N)], el_st)
        pltpu.sync_copy(erf.at[pl.ds(g * 2 * N, 2 * N)], er_st)

        def _zero_rows(r, _):
            for k in range(GW // 16):
                rows_b[0][r, pl.ds(16 * k, 16)] = zv
            wf_b[0][r, :] = zv
            return 0

        lax.fori_loop(0, C, _zero_rows, 0)
        for off, sz in ((0, 160), (160, 160), (320, 160), (480, 144)):
            pltpu.sync_copy(rows_b[0].at[pl.ds(0, sz)],
                            acc_sh.at[pl.ds(wid * ZB + off, sz)])
            pltpu.sync_copy(wf_b[0].at[pl.ds(0, sz)],
                            s_sh.at[pl.ds(wid * ZB + off, sz)])

        @pl.when(wid < 2)
        def _():
            pltpu.sync_copy(rows_b[0].at[pl.ds(0, 8)],
                            acc_sh.at[pl.ds(NT * ZB + wid * 8, 8)])
            pltpu.sync_copy(wf_b[0].at[pl.ds(0, 8)],
                            s_sh.at[pl.ds(NT * ZB + wid * 8, 8)])

        plsc.subcore_barrier()

        _prefetch(0, 0, g)

        def _pos(i, b, drain):
            if drain:
                _drain((b + 1) % 3)
            _prefetch(i + 1, (b + 1) % 3, g)
            _wait_gather(b)
            _scale(b)
            _scatter(b)

        _pos(0, 0, False)
        _pos(1, 1, False)

        def _triple(t, _):
            i = 3 * t + 2
            _pos(i, 2, True)
            _pos(i + 1, 0, True)
            _pos(i + 2, 1, True)
            return 0

        lax.fori_loop(0, (NCHUNK - 5) // 3, _triple, 0)
        _pos(NCHUNK - 3, (NCHUNK - 3) % 3, True)
        _pos(NCHUNK - 2, (NCHUNK - 2) % 3, True)
        lb = (NCHUNK - 1) % 3
        _drain((lb + 1) % 3)
        _wait_gather(lb)
        _scale(lb)
        _scatter(lb)
        _drain((lb + 2) % 3)
        _drain(lb)
        plsc.subcore_barrier()

        r0 = wid * ZB
        pltpu.sync_copy(acc_sh.at[pl.ds(r0, ZB)],
                        acc_out.at[c, p, pl.ds(r0, ZB)])
        pltpu.sync_copy(s_sh.at[pl.ds(r0, ZB)],
                        s_out.at[c, p, pl.ds(r0, ZB)])

        @pl.when(wid < 2)
        def _():
            r1 = NT * ZB + wid * 8
            pltpu.sync_copy(acc_sh.at[pl.ds(r1, 8)],
                            acc_out.at[c, p, pl.ds(r1, 8)])
            pltpu.sync_copy(s_sh.at[pl.ds(r1, 8)],
                            s_out.at[c, p, pl.ds(r1, 8)])

        plsc.subcore_barrier()


def _edge_call(featc, elf, erf, edges):
    mesh = plsc.VectorSubcoreMesh(core_axis_name="c", subcore_axis_name="s")
    fn = pl.kernel(
        _edge_body,
        out_type=[
            jax.ShapeDtypeStruct((2, 2, N, GW), jnp.float32),
            jax.ShapeDtypeStruct((2, 2, N, 16), jnp.float32),
        ],
        mesh=mesh,
        compiler_params=pltpu.CompilerParams(needs_layout_passes=False, use_tc_tiling_on_sc=False),
        scratch_types=[
            pltpu.VMEM((2 * N,), jnp.float32),
            pltpu.VMEM((2 * N,), jnp.float32),
            [pltpu.VMEM((2 * C,), jnp.int32)] * 3,
            [pltpu.VMEM((C,), jnp.int32)] * 3,
            [pltpu.VMEM((C,), jnp.int32)] * 3,
            [pltpu.VMEM((C, 16), jnp.float32)] * 3,
            [pltpu.VMEM((C, GW), jnp.float32)] * 3,
            pltpu.VMEM_SHARED((N, GW), jnp.float32),
            pltpu.VMEM_SHARED((N, 16), jnp.float32),
            [pltpu.SemaphoreType.DMA] * 3,
            [pltpu.SemaphoreType.DMA] * 3,
        ],
    )
    return fn(featc, elf, erf, edges)


def kernel(inputs, edge_index, W1, al1, ar1, b1, W2, al2, ar2, b2,
           W3, al3, ar3, b3, Wm, bm):
    edges = edge_index.T.reshape(2 * E)

    def edge(feat, el2, er2):
        return _edge_call(feat.reshape(4 * N, GW), el2.reshape(8 * N),
                          er2.reshape(8 * N), edges)

    feat, el2, er2 = _dense_pre(inputs, W1, al1.reshape(1, H * HID),
                                ar1.reshape(1, H * HID))
    acc, s = edge(feat, el2, er2)

    feat, el2, er2 = _dense_mid(acc, s, b1.reshape(1, H * HID), W2,
                                al2.reshape(1, H * HID), ar2.reshape(1, H * HID))
    acc, s = edge(feat, el2, er2)

    feat, el2, er2 = _dense_mid(acc, s, b2.reshape(1, H * HID), W3,
                                al3.reshape(1, H * HID), ar3.reshape(1, H * HID))
    acc, s = edge(feat, el2, er2)

    return _dense_fin(acc, s, b3.reshape(1, H * HID), Wm, bm.reshape(1, OUT))
